# baseline (speedup 1.0000x reference)
import functools

import jax
import jax.numpy as jnp
import numpy as np
from jax import lax
from jax.experimental import pallas as pl
from jax.experimental.pallas import tpu as pltpu
from jax.experimental.pallas import tpu_sc as plsc

NUM_FIELDS = 26
EMBED_DIM = 64
FIELD_SIZE = 100000
NUM_PAIRS = NUM_FIELDS * (NUM_FIELDS - 1) // 2

_NC, _NS = 2, 16
_NW = _NC * _NS
_PAIRS_PER_W = 16
_PAD_PAIRS = _NW * _PAIRS_PER_W


def _build_pair_consts() -> np.ndarray:
    posA = np.zeros(_PAD_PAIRS, np.int32)
    offA = np.zeros(_PAD_PAIRS, np.int32)
    posB = np.zeros(_PAD_PAIRS, np.int32)
    offB = np.zeros(_PAD_PAIRS, np.int32)
    p = 0
    for i in range(NUM_FIELDS - 1):
        for j in range(i + 1, NUM_FIELDS):
            posA[p], offA[p] = j * NUM_FIELDS + i, i * FIELD_SIZE
            posB[p], offB[p] = i * NUM_FIELDS + j, j * FIELD_SIZE
            p += 1
    const = np.zeros((_NW, 4, _PAIRS_PER_W), np.int32)
    for w in range(_NW):
        s = w * _PAIRS_PER_W
        const[w, 0] = posA[s:s + _PAIRS_PER_W]
        const[w, 1] = posB[s:s + _PAIRS_PER_W]
        const[w, 2] = offA[s:s + _PAIRS_PER_W]
        const[w, 3] = offB[s:s + _PAIRS_PER_W]
    return const.reshape(-1)


_PAIR_CONSTS = _build_pair_consts()


@functools.partial(
    pl.kernel,
    out_type=jax.ShapeDtypeStruct((_PAD_PAIRS, EMBED_DIM), jnp.float32),
    mesh=plsc.VectorSubcoreMesh(core_axis_name="c", subcore_axis_name="s"),
    compiler_params=pltpu.CompilerParams(use_tc_tiling_on_sc=False),
    scratch_types=[
        pltpu.VMEM((2 * _PAIRS_PER_W,), jnp.int32),
        pltpu.VMEM((2 * _PAIRS_PER_W,), jnp.int32),
        pltpu.VMEM((2 * _PAIRS_PER_W,), jnp.int32),
        pltpu.VMEM((2 * _PAIRS_PER_W,), jnp.int32),
        pltpu.VMEM((2 * _PAIRS_PER_W, EMBED_DIM), jnp.float32),
        pltpu.VMEM((_PAIRS_PER_W, EMBED_DIM), jnp.float32),
        pltpu.SemaphoreType.DMA,
    ],
)
def _ffm_sc(x26_hbm, const_hbm, table_hbm, out_hbm,
            pos_v, off_v, xval_v, idx_v, rows_v, prod_v, sem):
    wid = lax.axis_index("s") * _NC + lax.axis_index("c")
    base = wid * 4 * _PAIRS_PER_W
    pltpu.sync_copy(const_hbm.at[pl.ds(base, 2 * _PAIRS_PER_W)], pos_v)
    pltpu.sync_copy(const_hbm.at[pl.ds(base + 2 * _PAIRS_PER_W,
                                       2 * _PAIRS_PER_W)], off_v)
    pltpu.async_copy(x26_hbm.at[pos_v], xval_v, sem).wait()
    for h in range(2):
        sl = pl.ds(h * 16, 16)
        idx_v[sl] = xval_v[sl] + off_v[sl]
    pltpu.async_copy(table_hbm.at[idx_v], rows_v, sem).wait()
    for p in range(_PAIRS_PER_W):
        for c in range(EMBED_DIM // 16):
            sl = pl.ds(c * 16, 16)
            prod_v[p, sl] = rows_v[p, sl] * rows_v[_PAIRS_PER_W + p, sl]
    pltpu.sync_copy(prod_v, out_hbm.at[pl.ds(wid * _PAIRS_PER_W, _PAIRS_PER_W)])


def kernel(x, table):
    x26 = x[:NUM_FIELDS].astype(jnp.int32).reshape(-1)
    consts = jnp.asarray(_PAIR_CONSTS)
    res = _ffm_sc(x26, consts, table)
    return res[:NUM_PAIRS].T
